# Initial kernel scaffold; baseline (speedup 1.0000x reference)
import jax
import jax.numpy as jnp
from jax.experimental import pallas as pl


def kernel(x, cell_id, coordinates, connectivity):
    raise NotImplementedError("write your pallas kernel here")


# speedup vs baseline: 35.7480x; 35.7480x over previous
import functools

import jax
import jax.numpy as jnp
from jax import lax
from jax.experimental import pallas as pl
from jax.experimental.pallas import tpu as pltpu
from jax.experimental.pallas import tpu_sc as plsc

_LANES = 16


def _tec_kernel(n_pts, n_workers, chunk, n_nodes, n_cells,
                x_hbm, cid_hbm, coord_hbm, conn0_hbm, conn1_hbm, out_hbm,
                coord_v, conn0_v, conn1_v, x_v, cid_v, out_v):
    per_worker = n_pts // n_workers
    n_chunks = per_worker // chunk
    wid = lax.axis_index("s") * 2 + lax.axis_index("c")
    base = wid * per_worker

    pltpu.sync_copy(coord_hbm, coord_v)
    pltpu.sync_copy(conn0_hbm, conn0_v)
    pltpu.sync_copy(conn1_hbm, conn1_v)

    lane = lax.iota(jnp.int32, _LANES)

    def chunk_body(j, _):
        off = base + j * chunk
        pltpu.sync_copy(x_hbm.at[pl.ds(off, chunk)], x_v)
        pltpu.sync_copy(cid_hbm.at[pl.ds(off, chunk)], cid_v)

        def vec_body(k, _):
            o = k * _LANES
            cid = cid_v[pl.ds(o, _LANES)]
            n0 = plsc.load_gather(conn0_v, [cid])
            n1 = plsc.load_gather(conn1_v, [cid])
            x0 = plsc.load_gather(coord_v, [n0])
            x1 = plsc.load_gather(coord_v, [n1])
            xv = x_v[pl.ds(o, _LANES)]
            inv = 1.0 / (x0 - x1)
            na = (xv - x1) * inv
            nb = (x0 - xv) * inv
            pos = (o + lane) * 2
            plsc.store_scatter(out_v, [pos], na)
            plsc.store_scatter(out_v, [pos + 1], nb)
            return _

        lax.fori_loop(0, chunk // _LANES, vec_body, None, unroll=4)
        pltpu.sync_copy(out_v, out_hbm.at[pl.ds(2 * off, 2 * chunk)])
        return _

    lax.fori_loop(0, n_chunks, chunk_body, None)


def kernel(x, cell_id, coordinates, connectivity):
    n_pts = x.shape[0]
    n_nodes = coordinates.shape[0]
    n_cells = connectivity.shape[0]
    n_workers = 32
    chunk = 8192

    coord_flat = coordinates[:, 0]
    conn0 = connectivity[:, 0]
    conn1 = connectivity[:, 1]

    mesh = plsc.VectorSubcoreMesh(core_axis_name="c", subcore_axis_name="s")
    body = functools.partial(_tec_kernel, n_pts, n_workers, chunk,
                             n_nodes, n_cells)
    out_flat = pl.kernel(
        body,
        mesh=mesh,
        out_type=jax.ShapeDtypeStruct((2 * n_pts,), jnp.float32),
        compiler_params=pltpu.CompilerParams(needs_layout_passes=False),
        scratch_types=[
            pltpu.VMEM((n_nodes,), jnp.float32),
            pltpu.VMEM((n_cells,), jnp.int32),
            pltpu.VMEM((n_cells,), jnp.int32),
            pltpu.VMEM((chunk,), jnp.float32),
            pltpu.VMEM((chunk,), jnp.int32),
            pltpu.VMEM((2 * chunk,), jnp.float32),
        ],
    )(x, cell_id, coord_flat, conn0, conn1)
    return out_flat.reshape(n_pts, 2)
